# baseline (speedup 1.0000x reference)
import functools

import jax
import jax.numpy as jnp
from jax import lax
from jax.experimental import pallas as pl
from jax.experimental.pallas import tpu as pltpu
from jax.experimental.pallas import tpu_sc as plsc

N = 10000
E = 160000
D = 256
HD = 128
N_PAD = 10240
E_PAD = 163840
NC = 2
NS = 16
K = 128
KA = 32
ROWS_PER_SUB = N_PAD // NS
EDGES_PER_SUB_ALL = E_PAD // NS
EDGES_PER_SUB_HALF = E_PAD // 2 // NS

_mesh = plsc.VectorSubcoreMesh(core_axis_name="c", subcore_axis_name="s")
_f32 = jnp.float32


@functools.partial(
    pl.kernel,
    out_type=[
        jax.ShapeDtypeStruct((N_PAD, 16), _f32),
        jax.ShapeDtypeStruct((N_PAD, 16), _f32),
    ],
    mesh=_mesh,
    scratch_types=[
        pltpu.VMEM_SHARED((N_PAD, 16), _f32),
        pltpu.VMEM((K, 16), _f32),
        pltpu.VMEM((K, 16), _f32),
        pltpu.VMEM((K,), jnp.int32),
    ],
    compiler_params=pltpu.CompilerParams(use_tc_tiling_on_sc=False,
                                         needs_layout_passes=False),
)
def _deg_kernel(dst_hbm, deg0_hbm, deg1_hbm, acc, ones, zbuf, idx_d):
    cid = lax.axis_index("c")
    sid = lax.axis_index("s")

    def fill(i, _):
        ones[i, :] = jnp.ones((16,), _f32)
        zbuf[i, :] = jnp.zeros((16,), _f32)
        return 0

    lax.fori_loop(0, K, fill, 0)
    for r in range(ROWS_PER_SUB // K):
        pltpu.sync_copy(zbuf, acc.at[pl.ds((sid * 5 + r) * K, K)])
    plsc.subcore_barrier()

    base0 = cid * (E_PAD // 2) + sid * EDGES_PER_SUB_HALF

    def chunk(k, _):
        b = base0 + k * K
        pltpu.sync_copy(dst_hbm.at[pl.ds(b, K)], idx_d)
        pltpu.sync_copy(ones, acc.at[idx_d], add=True)
        return 0

    lax.fori_loop(0, EDGES_PER_SUB_HALF // K, chunk, 0)
    plsc.subcore_barrier()

    row0 = sid * ROWS_PER_SUB

    @pl.when(cid == 0)
    def _():
        pltpu.sync_copy(acc.at[pl.ds(row0, ROWS_PER_SUB)],
                        deg0_hbm.at[pl.ds(row0, ROWS_PER_SUB)])

    @pl.when(cid == 1)
    def _():
        pltpu.sync_copy(acc.at[pl.ds(row0, ROWS_PER_SUB)],
                        deg1_hbm.at[pl.ds(row0, ROWS_PER_SUB)])


@functools.partial(
    pl.kernel,
    out_type=[
        jax.ShapeDtypeStruct((N_PAD, HD), _f32),
        jax.ShapeDtypeStruct((N_PAD, HD), _f32),
    ],
    mesh=_mesh,
    scratch_types=[
        pltpu.VMEM_SHARED((N_PAD, HD), _f32),
        pltpu.VMEM((EDGES_PER_SUB_ALL,), jnp.int32),
        pltpu.VMEM((EDGES_PER_SUB_ALL,), jnp.int32),
        pltpu.VMEM((KA, HD), _f32),
        pltpu.VMEM((KA, HD), _f32),
        pltpu.VMEM((KA, HD), _f32),
        pltpu.VMEM((KA, HD), _f32),
        pltpu.SemaphoreType.DMA,
        pltpu.SemaphoreType.DMA,
        pltpu.SemaphoreType.DMA,
        pltpu.SemaphoreType.DMA,
    ],
    compiler_params=pltpu.CompilerParams(use_tc_tiling_on_sc=False,
                                         needs_layout_passes=False),
)
def _agg_kernel(t0_hbm, t1_hbm, src_hbm, dst_hbm, agg0_hbm, agg1_hbm,
                acc, srcb, dstb, r0, r1, r2, r3, gs0, gs1, gs2, gs3):
    cid = lax.axis_index("c")
    sid = lax.axis_index("s")
    rbufs = [r0, r1, r2, r3]
    gsems = [gs0, gs1, gs2, gs3]

    def fill(i, _):
        for v in range(HD // 16):
            r0[i, pl.ds(v * 16, 16)] = jnp.zeros((16,), _f32)
        return 0

    lax.fori_loop(0, KA, fill, 0)
    for r in range(ROWS_PER_SUB // KA):
        pltpu.sync_copy(r0, acc.at[pl.ds(sid * ROWS_PER_SUB + r * KA, KA)])
    plsc.subcore_barrier()

    base0 = sid * EDGES_PER_SUB_ALL
    pltpu.sync_copy(src_hbm.at[pl.ds(base0, EDGES_PER_SUB_ALL)], srcb)
    pltpu.sync_copy(dst_hbm.at[pl.ds(base0, EDGES_PER_SUB_ALL)], dstb)
    nchunk = EDGES_PER_SUB_ALL // KA
    niter = nchunk // 4

    def process(table):
        def start_g(c, s):
            pltpu.async_copy(table.at[srcb.at[pl.ds(c * KA, KA)]],
                             rbufs[s], gsems[s])

        def wait_g(c, s):
            pltpu.make_async_copy(table.at[srcb.at[pl.ds(c * KA, KA)]],
                                  rbufs[s], gsems[s]).wait()

        def addto(c, s):
            pltpu.sync_copy(rbufs[s], acc.at[dstb.at[pl.ds(c * KA, KA)]],
                            add=True)

        start_g(0, 0)
        start_g(1, 1)
        start_g(2, 2)

        def step(t, _):
            for s in range(4):
                c = 4 * t + s
                wait_g(c, s)
                if s == 0:
                    start_g(c + 3, (s + 3) % 4)
                else:
                    @pl.when(t < niter - 1)
                    def _():
                        start_g(c + 3, (s + 3) % 4)
                addto(c, s)
            return 0

        lax.fori_loop(0, niter, step, 0)

    @pl.when(cid == 0)
    def _():
        process(t0_hbm)

    @pl.when(cid == 1)
    def _():
        process(t1_hbm)

    plsc.subcore_barrier()

    row0 = sid * ROWS_PER_SUB

    @pl.when(cid == 0)
    def _():
        pltpu.sync_copy(acc.at[pl.ds(row0, ROWS_PER_SUB)],
                        agg0_hbm.at[pl.ds(row0, ROWS_PER_SUB)])

    @pl.when(cid == 1)
    def _():
        pltpu.sync_copy(acc.at[pl.ds(row0, ROWS_PER_SUB)],
                        agg1_hbm.at[pl.ds(row0, ROWS_PER_SUB)])


KE = 32


@functools.partial(
    pl.kernel,
    out_type=jax.ShapeDtypeStruct((E_PAD,), _f32),
    mesh=_mesh,
    scratch_types=[
        pltpu.VMEM((N_PAD,), _f32),
        pltpu.VMEM((N_PAD,), _f32),
        pltpu.VMEM((EDGES_PER_SUB_HALF,), jnp.int32),
        pltpu.VMEM((EDGES_PER_SUB_HALF,), jnp.int32),
        pltpu.VMEM((EDGES_PER_SUB_HALF,), _f32),
        pltpu.VMEM((EDGES_PER_SUB_HALF,), _f32),
        pltpu.VMEM((EDGES_PER_SUB_HALF,), _f32),
        pltpu.VMEM((KE, D), _f32),
        pltpu.VMEM((KE, D), _f32),
        pltpu.VMEM((KE, D), _f32),
        pltpu.VMEM((KE, D), _f32),
        pltpu.VMEM((KE, D), _f32),
        pltpu.VMEM((KE, D), _f32),
        pltpu.VMEM((KE, D), _f32),
        pltpu.VMEM((KE, D), _f32),
        pltpu.VMEM((3, 16), _f32),
        pltpu.SemaphoreType.DMA,
        pltpu.SemaphoreType.DMA,
        pltpu.SemaphoreType.DMA,
        pltpu.SemaphoreType.DMA,
    ],
    compiler_params=pltpu.CompilerParams(use_tc_tiling_on_sc=False,
                                         needs_layout_passes=False),
)
def _edge_kernel(zs_hbm, z_hbm, a_hbm, b_hbm, src_hbm, dst_hbm,
                 ea0_hbm, ea1_hbm, consts_hbm, y_hbm,
                 a_t, b_t, srcb, dstb, ea0b, ea1b, ybuf,
                 rs0, rt0, rs1, rt1, rs2, rt2, rs3, rt3,
                 cbuf, gs0, gs1, gs2, gs3):
    cid = lax.axis_index("c")
    sid = lax.axis_index("s")
    rsb = [rs0, rs1, rs2, rs3]
    rtb = [rt0, rt1, rt2, rt3]
    gsems = [gs0, gs1, gs2, gs3]

    base0 = (sid * NC + cid) * EDGES_PER_SUB_HALF
    pltpu.sync_copy(a_hbm, a_t)
    pltpu.sync_copy(b_hbm, b_t)
    pltpu.sync_copy(consts_hbm, cbuf)
    pltpu.sync_copy(src_hbm.at[pl.ds(base0, EDGES_PER_SUB_HALF)], srcb)
    pltpu.sync_copy(dst_hbm.at[pl.ds(base0, EDGES_PER_SUB_HALF)], dstb)
    pltpu.sync_copy(ea0_hbm.at[pl.ds(base0, EDGES_PER_SUB_HALF)], ea0b)
    pltpu.sync_copy(ea1_hbm.at[pl.ds(base0, EDGES_PER_SUB_HALF)], ea1b)
    w0 = cbuf[0, :]
    w1 = cbuf[1, :]
    bev = cbuf[2, :]
    lanes = lax.iota(jnp.int32, 16)
    nchunk = EDGES_PER_SUB_HALF // KE
    niter = nchunk // 4

    def start_g(c, s):
        pltpu.async_copy(zs_hbm.at[srcb.at[pl.ds(c * KE, KE)]], rsb[s],
                         gsems[s])
        pltpu.async_copy(z_hbm.at[dstb.at[pl.ds(c * KE, KE)]], rtb[s],
                         gsems[s])

    def wait_g(c, s):
        pltpu.make_async_copy(zs_hbm.at[srcb.at[pl.ds(c * KE, KE)]], rsb[s],
                              gsems[s]).wait()
        pltpu.make_async_copy(z_hbm.at[dstb.at[pl.ds(c * KE, KE)]], rtb[s],
                              gsems[s]).wait()

    def compute(c, s):
        rs = rsb[s]
        rt = rtb[s]
        for g in range(KE // 16):
            sl = pl.ds(c * KE + g * 16, 16)

            def edot(jj, vec):
                j = g * 16 + jj
                acc = rs[j, pl.ds(0, 16)] * rt[j, pl.ds(0, 16)]
                for v in range(1, D // 16):
                    acc = acc + rs[j, pl.ds(v * 16, 16)] * rt[j, pl.ds(v * 16, 16)]
                return jnp.where(lanes == jj, jnp.sum(acc), vec)

            pd16 = lax.fori_loop(0, 16, edot, jnp.zeros((16,), _f32))
            ia = srcb[sl]
            ib = dstb[sl]
            ga = plsc.load_gather(a_t, [ia])
            gb = plsc.load_gather(b_t, [ib])
            t = pd16 + ga + gb + w0 * ea0b[sl] + w1 * ea1b[sl] + bev
            ybuf[sl] = 1.0 / (1.0 + jnp.exp(-t))

    start_g(0, 0)
    start_g(1, 1)
    start_g(2, 2)

    def step(t, _):
        for s in range(4):
            c = 4 * t + s
            wait_g(c, s)
            if s == 0:
                start_g(c + 3, (s + 3) % 4)
            else:
                @pl.when(t < niter - 1)
                def _():
                    start_g(c + 3, (s + 3) % 4)
            compute(c, s)
        return 0

    lax.fori_loop(0, niter, step, 0)
    pltpu.sync_copy(ybuf, y_hbm.at[pl.ds(base0, EDGES_PER_SUB_HALF)])


BM = 512
_GRID = N_PAD // BM


def _mm1_body(x_ref, w_ref, d0_ref, d1_ref, hn0_ref, hn1_ref, dinv_ref):
    deg = 1.0 + d0_ref[:, 0:1] + d1_ref[:, 0:1]
    dinv = lax.rsqrt(deg)
    xw = jnp.dot(x_ref[...], w_ref[...], preferred_element_type=_f32)
    hn = xw * dinv
    hn0_ref[...] = hn[:, :HD]
    hn1_ref[...] = hn[:, HD:]
    dinv_ref[...] = dinv


def _mm1(x_p, W1, deg0, deg1):
    return pl.pallas_call(
        _mm1_body,
        grid=(_GRID,),
        in_specs=[
            pl.BlockSpec((BM, D), lambda i: (i, 0)),
            pl.BlockSpec((D, D), lambda i: (0, 0)),
            pl.BlockSpec((BM, 16), lambda i: (i, 0)),
            pl.BlockSpec((BM, 16), lambda i: (i, 0)),
        ],
        out_specs=[
            pl.BlockSpec((BM, HD), lambda i: (i, 0)),
            pl.BlockSpec((BM, HD), lambda i: (i, 0)),
            pl.BlockSpec((BM, 1), lambda i: (i, 0)),
        ],
        out_shape=[
            jax.ShapeDtypeStruct((N_PAD, HD), _f32),
            jax.ShapeDtypeStruct((N_PAD, HD), _f32),
            jax.ShapeDtypeStruct((N_PAD, 1), _f32),
        ],
    )(x_p, W1, deg0, deg1)


def _mm2_body(a0_ref, a1_ref, h0_ref, h1_ref, dinv_ref, b1_ref, w_ref,
              zn0_ref, zn1_ref):
    dinv = dinv_ref[...]
    agg = jnp.concatenate([a0_ref[...], a1_ref[...]], axis=1)
    hn = jnp.concatenate([h0_ref[...], h1_ref[...]], axis=1)
    h = jnp.maximum(dinv * (agg + hn) + b1_ref[...], 0.0)
    zw = jnp.dot(h, w_ref[...], preferred_element_type=_f32)
    zn = zw * dinv
    zn0_ref[...] = zn[:, :HD]
    zn1_ref[...] = zn[:, HD:]


def _mm2(agg0, agg1, hn0, hn1, dinv, b1_row, W2):
    return pl.pallas_call(
        _mm2_body,
        grid=(_GRID,),
        in_specs=[
            pl.BlockSpec((BM, HD), lambda i: (i, 0)),
            pl.BlockSpec((BM, HD), lambda i: (i, 0)),
            pl.BlockSpec((BM, HD), lambda i: (i, 0)),
            pl.BlockSpec((BM, HD), lambda i: (i, 0)),
            pl.BlockSpec((BM, 1), lambda i: (i, 0)),
            pl.BlockSpec((1, D), lambda i: (0, 0)),
            pl.BlockSpec((D, D), lambda i: (0, 0)),
        ],
        out_specs=[
            pl.BlockSpec((BM, HD), lambda i: (i, 0)),
            pl.BlockSpec((BM, HD), lambda i: (i, 0)),
        ],
        out_shape=[
            jax.ShapeDtypeStruct((N_PAD, HD), _f32),
            jax.ShapeDtypeStruct((N_PAD, HD), _f32),
        ],
    )(agg0, agg1, hn0, hn1, dinv, b1_row, W2)


def _zfin_body(a0_ref, a1_ref, z0_ref, z1_ref, dinv_ref, b2_ref,
               wp_ref, ws_ref, wt_ref, z_ref, zs_ref, a_ref, b_ref):
    dinv = dinv_ref[...]
    agg = jnp.concatenate([a0_ref[...], a1_ref[...]], axis=1)
    zn = jnp.concatenate([z0_ref[...], z1_ref[...]], axis=1)
    z = dinv * (agg + zn) + b2_ref[...]
    z_ref[...] = z
    zs_ref[...] = z * wp_ref[...]
    a_ref[...] = jnp.dot(z, ws_ref[...], preferred_element_type=_f32)
    b_ref[...] = jnp.dot(z, wt_ref[...], preferred_element_type=_f32)


def _zfin(agg0, agg1, zn0, zn1, dinv, b2_row, wp_row, ws_col, wt_col):
    return pl.pallas_call(
        _zfin_body,
        grid=(_GRID,),
        in_specs=[
            pl.BlockSpec((BM, HD), lambda i: (i, 0)),
            pl.BlockSpec((BM, HD), lambda i: (i, 0)),
            pl.BlockSpec((BM, HD), lambda i: (i, 0)),
            pl.BlockSpec((BM, HD), lambda i: (i, 0)),
            pl.BlockSpec((BM, 1), lambda i: (i, 0)),
            pl.BlockSpec((1, D), lambda i: (0, 0)),
            pl.BlockSpec((1, D), lambda i: (0, 0)),
            pl.BlockSpec((D, 1), lambda i: (0, 0)),
            pl.BlockSpec((D, 1), lambda i: (0, 0)),
        ],
        out_specs=[
            pl.BlockSpec((BM, D), lambda i: (i, 0)),
            pl.BlockSpec((BM, D), lambda i: (i, 0)),
            pl.BlockSpec((BM, 1), lambda i: (i, 0)),
            pl.BlockSpec((BM, 1), lambda i: (i, 0)),
        ],
        out_shape=[
            jax.ShapeDtypeStruct((N_PAD, D), _f32),
            jax.ShapeDtypeStruct((N_PAD, D), _f32),
            jax.ShapeDtypeStruct((N_PAD, 1), _f32),
            jax.ShapeDtypeStruct((N_PAD, 1), _f32),
        ],
    )(agg0, agg1, zn0, zn1, dinv, b2_row, wp_row, ws_col, wt_col)


def kernel(x, edge_index, edge_attr, W1, b1, W2, b2, We, be):
    src = edge_index[0]
    dst = edge_index[1]
    padlen = E_PAD - E
    pad_idx = jnp.full((padlen,), N, dtype=jnp.int32)
    src_p = jnp.concatenate([src, pad_idx])
    dst_p = jnp.concatenate([dst, pad_idx])
    ea0 = jnp.concatenate([edge_attr[:, 0], jnp.zeros((padlen,), _f32)])
    ea1 = jnp.concatenate([edge_attr[:, 1], jnp.zeros((padlen,), _f32)])
    x_p = jnp.pad(x, ((0, N_PAD - N), (0, 0)))

    b1_row = b1.reshape(1, D)
    b2_row = b2.reshape(1, D)
    wea = We[0:2, 0]
    ws_col = We[2:2 + D]
    wt_col = We[2 + D:2 + 2 * D]
    wp_row = We[2 + 2 * D:2 + 3 * D, 0].reshape(1, D)
    consts = jnp.stack([
        jnp.full((16,), wea[0], _f32),
        jnp.full((16,), wea[1], _f32),
        jnp.full((16,), be[0], _f32),
    ])

    deg0, deg1 = _deg_kernel(dst_p)
    hn0, hn1, dinv = _mm1(x_p, W1, deg0, deg1)
    agg10, agg11 = _agg_kernel(hn0, hn1, src_p, dst_p)
    zn0, zn1 = _mm2(agg10, agg11, hn0, hn1, dinv, b1_row, W2)
    agg20, agg21 = _agg_kernel(zn0, zn1, src_p, dst_p)
    z_t, zs_t, a_col, b_col = _zfin(agg20, agg21, zn0, zn1, dinv, b2_row,
                                    wp_row, ws_col, wt_col)
    y_pad = _edge_kernel(zs_t, z_t, a_col.reshape(N_PAD), b_col.reshape(N_PAD),
                         src_p, dst_p, ea0, ea1, consts)
    return y_pad[:E].reshape(E, 1)


# speedup vs baseline: 1.0021x; 1.0021x over previous
import functools

import jax
import jax.numpy as jnp
from jax import lax
from jax.experimental import pallas as pl
from jax.experimental.pallas import tpu as pltpu
from jax.experimental.pallas import tpu_sc as plsc

N = 10000
E = 160000
D = 256
HD = 128
N_PAD = 10240
E_PAD = 163840
NC = 2
NS = 16
K = 128
KA = 32
ROWS_PER_SUB = N_PAD // NS
EDGES_PER_SUB_ALL = E_PAD // NS
EDGES_PER_SUB_HALF = E_PAD // 2 // NS

_mesh = plsc.VectorSubcoreMesh(core_axis_name="c", subcore_axis_name="s")
_f32 = jnp.float32


@functools.partial(
    pl.kernel,
    out_type=[
        jax.ShapeDtypeStruct((N_PAD, 16), _f32),
        jax.ShapeDtypeStruct((N_PAD, 16), _f32),
    ],
    mesh=_mesh,
    scratch_types=[
        pltpu.VMEM_SHARED((N_PAD, 16), _f32),
        pltpu.VMEM((K, 16), _f32),
        pltpu.VMEM((K, 16), _f32),
        pltpu.VMEM((K,), jnp.int32),
    ],
    compiler_params=pltpu.CompilerParams(use_tc_tiling_on_sc=False,
                                         needs_layout_passes=False),
)
def _deg_kernel(dst_hbm, deg0_hbm, deg1_hbm, acc, ones, zbuf, idx_d):
    cid = lax.axis_index("c")
    sid = lax.axis_index("s")

    def fill(i, _):
        ones[i, :] = jnp.ones((16,), _f32)
        zbuf[i, :] = jnp.zeros((16,), _f32)
        return 0

    lax.fori_loop(0, K, fill, 0)
    for r in range(ROWS_PER_SUB // K):
        pltpu.sync_copy(zbuf, acc.at[pl.ds((sid * 5 + r) * K, K)])
    plsc.subcore_barrier()

    base0 = cid * (E_PAD // 2) + sid * EDGES_PER_SUB_HALF

    def chunk(k, _):
        b = base0 + k * K
        pltpu.sync_copy(dst_hbm.at[pl.ds(b, K)], idx_d)
        pltpu.sync_copy(ones, acc.at[idx_d], add=True)
        return 0

    lax.fori_loop(0, EDGES_PER_SUB_HALF // K, chunk, 0)
    plsc.subcore_barrier()

    row0 = sid * ROWS_PER_SUB

    @pl.when(cid == 0)
    def _():
        pltpu.sync_copy(acc.at[pl.ds(row0, ROWS_PER_SUB)],
                        deg0_hbm.at[pl.ds(row0, ROWS_PER_SUB)])

    @pl.when(cid == 1)
    def _():
        pltpu.sync_copy(acc.at[pl.ds(row0, ROWS_PER_SUB)],
                        deg1_hbm.at[pl.ds(row0, ROWS_PER_SUB)])


@functools.partial(
    pl.kernel,
    out_type=[
        jax.ShapeDtypeStruct((N_PAD, HD), _f32),
        jax.ShapeDtypeStruct((N_PAD, HD), _f32),
    ],
    mesh=_mesh,
    scratch_types=[
        pltpu.VMEM_SHARED((N_PAD, HD), _f32),
        pltpu.VMEM((EDGES_PER_SUB_ALL,), jnp.int32),
        pltpu.VMEM((EDGES_PER_SUB_ALL,), jnp.int32),
        pltpu.VMEM((KA, HD), _f32),
        pltpu.VMEM((KA, HD), _f32),
        pltpu.VMEM((KA, HD), _f32),
        pltpu.VMEM((KA, HD), _f32),
        pltpu.SemaphoreType.DMA,
        pltpu.SemaphoreType.DMA,
        pltpu.SemaphoreType.DMA,
        pltpu.SemaphoreType.DMA,
    ],
    compiler_params=pltpu.CompilerParams(use_tc_tiling_on_sc=False,
                                         needs_layout_passes=False),
)
def _agg_kernel(t0_hbm, t1_hbm, src_hbm, dst_hbm, agg0_hbm, agg1_hbm,
                acc, srcb, dstb, r0, r1, r2, r3, gs0, gs1, gs2, gs3):
    cid = lax.axis_index("c")
    sid = lax.axis_index("s")
    rbufs = [r0, r1, r2, r3]
    gsems = [gs0, gs1, gs2, gs3]

    def fill(i, _):
        for v in range(HD // 16):
            r0[i, pl.ds(v * 16, 16)] = jnp.zeros((16,), _f32)
        return 0

    lax.fori_loop(0, KA, fill, 0)
    for r in range(ROWS_PER_SUB // KA):
        pltpu.sync_copy(r0, acc.at[pl.ds(sid * ROWS_PER_SUB + r * KA, KA)])
    plsc.subcore_barrier()

    base0 = sid * EDGES_PER_SUB_ALL
    pltpu.sync_copy(src_hbm.at[pl.ds(base0, EDGES_PER_SUB_ALL)], srcb)
    pltpu.sync_copy(dst_hbm.at[pl.ds(base0, EDGES_PER_SUB_ALL)], dstb)
    nchunk = EDGES_PER_SUB_ALL // KA
    niter = nchunk // 4

    def process(table):
        def start_g(c, s):
            pltpu.async_copy(table.at[srcb.at[pl.ds(c * KA, KA)]],
                             rbufs[s], gsems[s])

        def wait_g(c, s):
            pltpu.make_async_copy(table.at[srcb.at[pl.ds(c * KA, KA)]],
                                  rbufs[s], gsems[s]).wait()

        def addto(c, s):
            pltpu.sync_copy(rbufs[s], acc.at[dstb.at[pl.ds(c * KA, KA)]],
                            add=True)

        start_g(0, 0)
        start_g(1, 1)
        start_g(2, 2)

        def step(t, _):
            for s in range(4):
                c = 4 * t + s
                wait_g(c, s)
                if s == 0:
                    start_g(c + 3, (s + 3) % 4)
                else:
                    @pl.when(t < niter - 1)
                    def _():
                        start_g(c + 3, (s + 3) % 4)
                addto(c, s)
            return 0

        lax.fori_loop(0, niter, step, 0)

    @pl.when(cid == 0)
    def _():
        process(t0_hbm)

    @pl.when(cid == 1)
    def _():
        process(t1_hbm)

    plsc.subcore_barrier()

    row0 = sid * ROWS_PER_SUB

    @pl.when(cid == 0)
    def _():
        pltpu.sync_copy(acc.at[pl.ds(row0, ROWS_PER_SUB)],
                        agg0_hbm.at[pl.ds(row0, ROWS_PER_SUB)])

    @pl.when(cid == 1)
    def _():
        pltpu.sync_copy(acc.at[pl.ds(row0, ROWS_PER_SUB)],
                        agg1_hbm.at[pl.ds(row0, ROWS_PER_SUB)])


KE = 32
T0 = 108
T1 = 212
EMAX = max(T0, T1) * KE


@functools.partial(
    pl.kernel,
    out_type=jax.ShapeDtypeStruct((E_PAD,), _f32),
    mesh=_mesh,
    scratch_types=[
        pltpu.VMEM((N_PAD,), _f32),
        pltpu.VMEM((N_PAD,), _f32),
        pltpu.VMEM((EMAX,), jnp.int32),
        pltpu.VMEM((EMAX,), jnp.int32),
        pltpu.VMEM((EMAX,), _f32),
        pltpu.VMEM((EMAX,), _f32),
        pltpu.VMEM((EMAX,), _f32),
        pltpu.VMEM((KE, D), _f32),
        pltpu.VMEM((KE, D), _f32),
        pltpu.VMEM((KE, D), _f32),
        pltpu.VMEM((KE, D), _f32),
        pltpu.VMEM((KE, D), _f32),
        pltpu.VMEM((KE, D), _f32),
        pltpu.VMEM((KE, D), _f32),
        pltpu.VMEM((KE, D), _f32),
        pltpu.VMEM((3, 16), _f32),
        pltpu.SemaphoreType.DMA,
        pltpu.SemaphoreType.DMA,
        pltpu.SemaphoreType.DMA,
        pltpu.SemaphoreType.DMA,
    ],
    compiler_params=pltpu.CompilerParams(use_tc_tiling_on_sc=False,
                                         needs_layout_passes=False),
)
def _edge_kernel(zs_hbm, z_hbm, a_hbm, b_hbm, src_hbm, dst_hbm,
                 ea0_hbm, ea1_hbm, consts_hbm, y_hbm,
                 a_t, b_t, srcb, dstb, ea0b, ea1b, ybuf,
                 rs0, rt0, rs1, rt1, rs2, rt2, rs3, rt3,
                 cbuf, gs0, gs1, gs2, gs3):
    cid = lax.axis_index("c")
    sid = lax.axis_index("s")
    rsb = [rs0, rs1, rs2, rs3]
    rtb = [rt0, rt1, rt2, rt3]
    gsems = [gs0, gs1, gs2, gs3]

    nsub0 = T0 * KE
    nsub1 = T1 * KE
    base0 = jnp.where(cid == 0, sid * nsub0, NS * nsub0 + sid * nsub1)
    nmine = jnp.where(cid == 0, nsub0, nsub1)
    pltpu.sync_copy(a_hbm, a_t)
    pltpu.sync_copy(b_hbm, b_t)
    pltpu.sync_copy(consts_hbm, cbuf)
    w0 = cbuf[0, :]
    w1 = cbuf[1, :]
    bev = cbuf[2, :]
    lanes = lax.iota(jnp.int32, 16)

    def start_g(c, s):
        pltpu.async_copy(zs_hbm.at[srcb.at[pl.ds(c * KE, KE)]], rsb[s],
                         gsems[s])
        pltpu.async_copy(z_hbm.at[dstb.at[pl.ds(c * KE, KE)]], rtb[s],
                         gsems[s])

    def wait_g(c, s):
        pltpu.make_async_copy(zs_hbm.at[srcb.at[pl.ds(c * KE, KE)]], rsb[s],
                              gsems[s]).wait()
        pltpu.make_async_copy(z_hbm.at[dstb.at[pl.ds(c * KE, KE)]], rtb[s],
                              gsems[s]).wait()

    def compute(c, s):
        rs = rsb[s]
        rt = rtb[s]
        for g in range(KE // 16):
            sl = pl.ds(c * KE + g * 16, 16)

            def edot(jj, vec):
                j = g * 16 + jj
                acc = rs[j, pl.ds(0, 16)] * rt[j, pl.ds(0, 16)]
                for v in range(1, D // 16):
                    acc = acc + rs[j, pl.ds(v * 16, 16)] * rt[j, pl.ds(v * 16, 16)]
                return jnp.where(lanes == jj, jnp.sum(acc), vec)

            pd16 = lax.fori_loop(0, 16, edot, jnp.zeros((16,), _f32))
            ia = srcb[sl]
            ib = dstb[sl]
            ga = plsc.load_gather(a_t, [ia])
            gb = plsc.load_gather(b_t, [ib])
            t = pd16 + ga + gb + w0 * ea0b[sl] + w1 * ea1b[sl] + bev
            ybuf[sl] = 1.0 / (1.0 + jnp.exp(-t))

    def run(nchunk):
        niter = nchunk // 4

        start_g(0, 0)
        start_g(1, 1)
        start_g(2, 2)

        def step(t, _):
            for s in range(4):
                c = 4 * t + s
                wait_g(c, s)
                if s == 0:
                    start_g(c + 3, (s + 3) % 4)
                else:
                    @pl.when(t < niter - 1)
                    def _():
                        start_g(c + 3, (s + 3) % 4)
                compute(c, s)
            return 0

        lax.fori_loop(0, niter, step, 0)

    @pl.when(cid == 0)
    def _():
        pltpu.sync_copy(src_hbm.at[pl.ds(base0, nsub0)], srcb.at[pl.ds(0, nsub0)])
        pltpu.sync_copy(dst_hbm.at[pl.ds(base0, nsub0)], dstb.at[pl.ds(0, nsub0)])
        pltpu.sync_copy(ea0_hbm.at[pl.ds(base0, nsub0)], ea0b.at[pl.ds(0, nsub0)])
        pltpu.sync_copy(ea1_hbm.at[pl.ds(base0, nsub0)], ea1b.at[pl.ds(0, nsub0)])
        run(T0)
        pltpu.sync_copy(ybuf.at[pl.ds(0, nsub0)], y_hbm.at[pl.ds(base0, nsub0)])

    @pl.when(cid == 1)
    def _():
        pltpu.sync_copy(src_hbm.at[pl.ds(base0, nsub1)], srcb.at[pl.ds(0, nsub1)])
        pltpu.sync_copy(dst_hbm.at[pl.ds(base0, nsub1)], dstb.at[pl.ds(0, nsub1)])
        pltpu.sync_copy(ea0_hbm.at[pl.ds(base0, nsub1)], ea0b.at[pl.ds(0, nsub1)])
        pltpu.sync_copy(ea1_hbm.at[pl.ds(base0, nsub1)], ea1b.at[pl.ds(0, nsub1)])
        run(T1)
        pltpu.sync_copy(ybuf.at[pl.ds(0, nsub1)], y_hbm.at[pl.ds(base0, nsub1)])


BM = 512
_GRID = N_PAD // BM


def _mm1_body(x_ref, w_ref, d0_ref, d1_ref, hn0_ref, hn1_ref, dinv_ref):
    deg = 1.0 + d0_ref[:, 0:1] + d1_ref[:, 0:1]
    dinv = lax.rsqrt(deg)
    xw = jnp.dot(x_ref[...], w_ref[...], preferred_element_type=_f32)
    hn = xw * dinv
    hn0_ref[...] = hn[:, :HD]
    hn1_ref[...] = hn[:, HD:]
    dinv_ref[...] = dinv


def _mm1(x_p, W1, deg0, deg1):
    return pl.pallas_call(
        _mm1_body,
        grid=(_GRID,),
        in_specs=[
            pl.BlockSpec((BM, D), lambda i: (i, 0)),
            pl.BlockSpec((D, D), lambda i: (0, 0)),
            pl.BlockSpec((BM, 16), lambda i: (i, 0)),
            pl.BlockSpec((BM, 16), lambda i: (i, 0)),
        ],
        out_specs=[
            pl.BlockSpec((BM, HD), lambda i: (i, 0)),
            pl.BlockSpec((BM, HD), lambda i: (i, 0)),
            pl.BlockSpec((BM, 1), lambda i: (i, 0)),
        ],
        out_shape=[
            jax.ShapeDtypeStruct((N_PAD, HD), _f32),
            jax.ShapeDtypeStruct((N_PAD, HD), _f32),
            jax.ShapeDtypeStruct((N_PAD, 1), _f32),
        ],
    )(x_p, W1, deg0, deg1)


def _mm2_body(a0_ref, a1_ref, h0_ref, h1_ref, dinv_ref, b1_ref, w_ref,
              zn0_ref, zn1_ref):
    dinv = dinv_ref[...]
    agg = jnp.concatenate([a0_ref[...], a1_ref[...]], axis=1)
    hn = jnp.concatenate([h0_ref[...], h1_ref[...]], axis=1)
    h = jnp.maximum(dinv * (agg + hn) + b1_ref[...], 0.0)
    zw = jnp.dot(h, w_ref[...], preferred_element_type=_f32)
    zn = zw * dinv
    zn0_ref[...] = zn[:, :HD]
    zn1_ref[...] = zn[:, HD:]


def _mm2(agg0, agg1, hn0, hn1, dinv, b1_row, W2):
    return pl.pallas_call(
        _mm2_body,
        grid=(_GRID,),
        in_specs=[
            pl.BlockSpec((BM, HD), lambda i: (i, 0)),
            pl.BlockSpec((BM, HD), lambda i: (i, 0)),
            pl.BlockSpec((BM, HD), lambda i: (i, 0)),
            pl.BlockSpec((BM, HD), lambda i: (i, 0)),
            pl.BlockSpec((BM, 1), lambda i: (i, 0)),
            pl.BlockSpec((1, D), lambda i: (0, 0)),
            pl.BlockSpec((D, D), lambda i: (0, 0)),
        ],
        out_specs=[
            pl.BlockSpec((BM, HD), lambda i: (i, 0)),
            pl.BlockSpec((BM, HD), lambda i: (i, 0)),
        ],
        out_shape=[
            jax.ShapeDtypeStruct((N_PAD, HD), _f32),
            jax.ShapeDtypeStruct((N_PAD, HD), _f32),
        ],
    )(agg0, agg1, hn0, hn1, dinv, b1_row, W2)


def _zfin_body(a0_ref, a1_ref, z0_ref, z1_ref, dinv_ref, b2_ref,
               wp_ref, ws_ref, wt_ref, z_ref, zs_ref, a_ref, b_ref):
    dinv = dinv_ref[...]
    agg = jnp.concatenate([a0_ref[...], a1_ref[...]], axis=1)
    zn = jnp.concatenate([z0_ref[...], z1_ref[...]], axis=1)
    z = dinv * (agg + zn) + b2_ref[...]
    z_ref[...] = z
    zs_ref[...] = z * wp_ref[...]
    a_ref[...] = jnp.dot(z, ws_ref[...], preferred_element_type=_f32)
    b_ref[...] = jnp.dot(z, wt_ref[...], preferred_element_type=_f32)


def _zfin(agg0, agg1, zn0, zn1, dinv, b2_row, wp_row, ws_col, wt_col):
    return pl.pallas_call(
        _zfin_body,
        grid=(_GRID,),
        in_specs=[
            pl.BlockSpec((BM, HD), lambda i: (i, 0)),
            pl.BlockSpec((BM, HD), lambda i: (i, 0)),
            pl.BlockSpec((BM, HD), lambda i: (i, 0)),
            pl.BlockSpec((BM, HD), lambda i: (i, 0)),
            pl.BlockSpec((BM, 1), lambda i: (i, 0)),
            pl.BlockSpec((1, D), lambda i: (0, 0)),
            pl.BlockSpec((1, D), lambda i: (0, 0)),
            pl.BlockSpec((D, 1), lambda i: (0, 0)),
            pl.BlockSpec((D, 1), lambda i: (0, 0)),
        ],
        out_specs=[
            pl.BlockSpec((BM, D), lambda i: (i, 0)),
            pl.BlockSpec((BM, D), lambda i: (i, 0)),
            pl.BlockSpec((BM, 1), lambda i: (i, 0)),
            pl.BlockSpec((BM, 1), lambda i: (i, 0)),
        ],
        out_shape=[
            jax.ShapeDtypeStruct((N_PAD, D), _f32),
            jax.ShapeDtypeStruct((N_PAD, D), _f32),
            jax.ShapeDtypeStruct((N_PAD, 1), _f32),
            jax.ShapeDtypeStruct((N_PAD, 1), _f32),
        ],
    )(agg0, agg1, zn0, zn1, dinv, b2_row, wp_row, ws_col, wt_col)


def kernel(x, edge_index, edge_attr, W1, b1, W2, b2, We, be):
    src = edge_index[0]
    dst = edge_index[1]
    padlen = E_PAD - E
    pad_idx = jnp.full((padlen,), N, dtype=jnp.int32)
    src_p = jnp.concatenate([src, pad_idx])
    dst_p = jnp.concatenate([dst, pad_idx])
    ea0 = jnp.concatenate([edge_attr[:, 0], jnp.zeros((padlen,), _f32)])
    ea1 = jnp.concatenate([edge_attr[:, 1], jnp.zeros((padlen,), _f32)])
    x_p = jnp.pad(x, ((0, N_PAD - N), (0, 0)))

    b1_row = b1.reshape(1, D)
    b2_row = b2.reshape(1, D)
    wea = We[0:2, 0]
    ws_col = We[2:2 + D]
    wt_col = We[2 + D:2 + 2 * D]
    wp_row = We[2 + 2 * D:2 + 3 * D, 0].reshape(1, D)
    consts = jnp.stack([
        jnp.full((16,), wea[0], _f32),
        jnp.full((16,), wea[1], _f32),
        jnp.full((16,), be[0], _f32),
    ])

    deg0, deg1 = _deg_kernel(dst_p)
    hn0, hn1, dinv = _mm1(x_p, W1, deg0, deg1)
    agg10, agg11 = _agg_kernel(hn0, hn1, src_p, dst_p)
    zn0, zn1 = _mm2(agg10, agg11, hn0, hn1, dinv, b1_row, W2)
    agg20, agg21 = _agg_kernel(zn0, zn1, src_p, dst_p)
    z_t, zs_t, a_col, b_col = _zfin(agg20, agg21, zn0, zn1, dinv, b2_row,
                                    wp_row, ws_col, wt_col)
    y_pad = _edge_kernel(zs_t, z_t, a_col.reshape(N_PAD), b_col.reshape(N_PAD),
                         src_p, dst_p, ea0, ea1, consts)
    return y_pad[:E].reshape(E, 1)
